# baseline (speedup 1.0000x reference)
import jax
import jax.numpy as jnp
from jax.experimental import pallas as pl
from jax.experimental.pallas import tpu as pltpu

D_MODEL = 1024
TIME_STEPS = 16
MAX_LATENCY = 10
BS = 256


def _body(x_ref, out_ref, t_scratch):
    t_idx = pl.program_id(2)

    @pl.when(t_idx == 0)
    def _():
        t_scratch[...] = jnp.round(
            MAX_LATENCY * (1.0 - jax.nn.sigmoid(x_ref[0])))

    tv = t_idx.astype(jnp.float32)
    out_ref[0, 0] = jnp.where(t_scratch[...] == tv, 1.0, 0.0).astype(jnp.float32)


def kernel(x):
    B, S, D = x.shape
    grid = (B, S // BS, TIME_STEPS)
    return pl.pallas_call(
        _body,
        grid=grid,
        in_specs=[pl.BlockSpec((1, BS, D), lambda b, s, t: (b, s, 0))],
        out_specs=pl.BlockSpec((1, 1, BS, D), lambda b, s, t: (b, t, s, 0)),
        out_shape=jax.ShapeDtypeStruct((B, TIME_STEPS, S, D), jnp.float32),
        scratch_shapes=[pltpu.VMEM((BS, D), jnp.float32)],
    )(x)


# speedup vs baseline: 1.8185x; 1.8185x over previous
import jax
import jax.numpy as jnp
from jax.experimental import pallas as pl

D_MODEL = 1024
TIME_STEPS = 16
MAX_LATENCY = 10
BS = 256


def _body(x_ref, out_ref):
    xv = x_ref[0]
    t = jnp.round(MAX_LATENCY * (1.0 - jax.nn.sigmoid(xv)))
    for k in range(TIME_STEPS):
        out_ref[0, k] = jnp.where(t == jnp.float32(k), 1.0, 0.0).astype(jnp.float32)


def kernel(x):
    B, S, D = x.shape
    grid = (B, S // BS)
    return pl.pallas_call(
        _body,
        grid=grid,
        in_specs=[pl.BlockSpec((1, BS, D), lambda b, s: (b, s, 0))],
        out_specs=pl.BlockSpec((1, TIME_STEPS, BS, D), lambda b, s: (b, 0, s, 0)),
        out_shape=jax.ShapeDtypeStruct((B, TIME_STEPS, S, D), jnp.float32),
    )(x)
